# Initial kernel scaffold; baseline (speedup 1.0000x reference)
import jax
import jax.numpy as jnp
from jax.experimental import pallas as pl


def kernel(image, lut):
    raise NotImplementedError("write your pallas kernel here")


# speedup vs baseline: 1648.1284x; 1648.1284x over previous
import functools

import jax
import jax.numpy as jnp
from jax import lax
from jax.experimental import pallas as pl
from jax.experimental.pallas import tpu as pltpu
from jax.experimental.pallas import tpu_sc as plsc

_NUM_CORES = 2
_NUM_SUBCORES = 16
_LANES = 16
_NW = _NUM_CORES * _NUM_SUBCORES

_B = 8
_C = 3
_N = 512 * 512
_D = 33
_LUT_CH = _D * _D * _D
_LUT_ROW = _C * _LUT_CH
_LUT_ROW_PAD = 107816

_WPB = _NW // _B
_PPW = _N // _WPB
_CHUNK = 2048
_NVEC = _CHUNK // _LANES
_NCHUNK = _PPW // _CHUNK


def _lut_body(img_hbm, lut_hbm, out_hbm, lut_v, in_v, out_v):
    cid = lax.axis_index("c")
    sid = lax.axis_index("s")
    wid = sid * _NUM_CORES + cid
    b = wid // _WPB
    w = wid % _WPB

    pltpu.sync_copy(lut_hbm.at[pl.ds(b * _LUT_ROW_PAD, _LUT_ROW_PAD)], lut_v)

    @pl.loop(0, _NCHUNK)
    def _chunk(i):
        off = b * _C * _N + w * _PPW + i * _CHUNK
        for ch in range(_C):
            pltpu.sync_copy(img_hbm.at[pl.ds(off + ch * _N, _CHUNK)],
                            in_v.at[pl.ds(ch * _CHUNK, _CHUNK)])

        @pl.loop(0, _NVEC)
        def _vec(j):
            p = j * _LANES
            r = in_v[pl.ds(p, _LANES)]
            g = in_v[pl.ds(_CHUNK + p, _LANES)]
            bl = in_v[pl.ds(2 * _CHUNK + p, _LANES)]
            x = r * 32.0
            y = g * 32.0
            z = bl * 32.0
            x0 = x.astype(jnp.int32)
            y0 = y.astype(jnp.int32)
            z0 = z.astype(jnp.int32)
            wx = x - x0.astype(jnp.float32)
            wy = y - y0.astype(jnp.float32)
            wz = z - z0.astype(jnp.float32)
            idx = (z0 * _D + y0) * _D + x0
            i000 = idx
            i001 = idx + 1
            i010 = idx + _D
            i011 = idx + (_D + 1)
            i100 = idx + _D * _D
            i101 = idx + (_D * _D + 1)
            i110 = idx + (_D * _D + _D)
            i111 = idx + (_D * _D + _D + 1)
            ux = 1.0 - wx
            uy = 1.0 - wy
            uz = 1.0 - wz
            a00 = uz * uy
            a01 = uz * wy
            a10 = wz * uy
            a11 = wz * wy
            w000 = a00 * ux
            w001 = a00 * wx
            w010 = a01 * ux
            w011 = a01 * wx
            w100 = a10 * ux
            w101 = a10 * wx
            w110 = a11 * ux
            w111 = a11 * wx
            corners = (
                (i000, w000), (i001, w001), (i010, w010), (i011, w011),
                (i100, w100), (i101, w101), (i110, w110), (i111, w111),
            )
            for ch in range(_C):
                o = ch * _LUT_CH
                acc = None
                for iv, wv in corners:
                    gi = iv if o == 0 else iv + o
                    v = plsc.load_gather(lut_v, [gi]) * wv
                    acc = v if acc is None else acc + v
                out_v[pl.ds(ch * _CHUNK + p, _LANES)] = acc

        for ch in range(_C):
            pltpu.sync_copy(out_v.at[pl.ds(ch * _CHUNK, _CHUNK)],
                            out_hbm.at[pl.ds(off + ch * _N, _CHUNK)])


_lut_apply = functools.partial(
    pl.kernel,
    out_type=jax.ShapeDtypeStruct((_B * _C * _N,), jnp.float32),
    mesh=plsc.VectorSubcoreMesh(
        core_axis_name="c", subcore_axis_name="s",
        num_cores=_NUM_CORES, num_subcores=_NUM_SUBCORES),
    compiler_params=pltpu.CompilerParams(needs_layout_passes=False),
    scratch_types=[
        pltpu.VMEM((_LUT_ROW_PAD,), jnp.float32),
        pltpu.VMEM((_C * _CHUNK,), jnp.float32),
        pltpu.VMEM((_C * _CHUNK,), jnp.float32),
    ],
)(_lut_body)


def kernel(image, lut):
    B, C, H, W = image.shape
    img = image.reshape(B * C * H * W)
    lut_flat = lut.reshape(B, _LUT_ROW)
    lut_pad = jnp.pad(lut_flat, ((0, 0), (0, _LUT_ROW_PAD - _LUT_ROW)))
    out = _lut_apply(img, lut_pad.reshape(-1))
    return out.reshape(B, C, H, W)


# speedup vs baseline: 1995.9515x; 1.2110x over previous
import functools

import jax
import jax.numpy as jnp
from jax import lax
from jax.experimental import pallas as pl
from jax.experimental.pallas import tpu as pltpu
from jax.experimental.pallas import tpu_sc as plsc

_NUM_CORES = 2
_NUM_SUBCORES = 16
_LANES = 16
_NW = _NUM_CORES * _NUM_SUBCORES

_B = 8
_C = 3
_N = 512 * 512
_D = 33
_LUT_CH = _D * _D * _D
_LUT_ROW = _C * _LUT_CH
_LUT_ROW_PAD = 107816

_WPB = _NW // _B
_PPW = _N // _WPB
_CHUNK = 2048
_NVEC = _CHUNK // _LANES
_NCHUNK = _PPW // _CHUNK


def _lut_body(img_hbm, lut_hbm, out_hbm, lut_v, in_v, out_v):
    cid = lax.axis_index("c")
    sid = lax.axis_index("s")
    wid = sid * _NUM_CORES + cid
    b = wid // _WPB
    w = wid % _WPB

    pltpu.sync_copy(lut_hbm.at[pl.ds(b * _LUT_ROW_PAD, _LUT_ROW_PAD)], lut_v)

    @pl.loop(0, _NCHUNK)
    def _chunk(i):
        off = b * _C * _N + w * _PPW + i * _CHUNK
        for ch in range(_C):
            pltpu.sync_copy(img_hbm.at[pl.ds(off + ch * _N, _CHUNK)],
                            in_v.at[pl.ds(ch * _CHUNK, _CHUNK)])

        @plsc.parallel_loop(0, _NVEC, 1, unroll=2)
        def _vec(j):
            p = j * _LANES
            r = in_v[pl.ds(p, _LANES)]
            g = in_v[pl.ds(_CHUNK + p, _LANES)]
            bl = in_v[pl.ds(2 * _CHUNK + p, _LANES)]
            x = r * 32.0
            y = g * 32.0
            z = bl * 32.0
            x0 = x.astype(jnp.int32)
            y0 = y.astype(jnp.int32)
            z0 = z.astype(jnp.int32)
            wx = x - x0.astype(jnp.float32)
            wy = y - y0.astype(jnp.float32)
            wz = z - z0.astype(jnp.float32)
            idx = (z0 * _D + y0) * _D + x0
            i000 = idx
            i001 = idx + 1
            i010 = idx + _D
            i011 = idx + (_D + 1)
            i100 = idx + _D * _D
            i101 = idx + (_D * _D + 1)
            i110 = idx + (_D * _D + _D)
            i111 = idx + (_D * _D + _D + 1)
            ux = 1.0 - wx
            uy = 1.0 - wy
            uz = 1.0 - wz
            a00 = uz * uy
            a01 = uz * wy
            a10 = wz * uy
            a11 = wz * wy
            w000 = a00 * ux
            w001 = a00 * wx
            w010 = a01 * ux
            w011 = a01 * wx
            w100 = a10 * ux
            w101 = a10 * wx
            w110 = a11 * ux
            w111 = a11 * wx
            corners = (
                (i000, w000), (i001, w001), (i010, w010), (i011, w011),
                (i100, w100), (i101, w101), (i110, w110), (i111, w111),
            )
            for ch in range(_C):
                o = ch * _LUT_CH
                t = [plsc.load_gather(lut_v, [iv if o == 0 else iv + o]) * wv
                     for iv, wv in corners]
                acc = (((t[0] + t[1]) + (t[2] + t[3]))
                       + ((t[4] + t[5]) + (t[6] + t[7])))
                out_v[pl.ds(ch * _CHUNK + p, _LANES)] = acc

        for ch in range(_C):
            pltpu.sync_copy(out_v.at[pl.ds(ch * _CHUNK, _CHUNK)],
                            out_hbm.at[pl.ds(off + ch * _N, _CHUNK)])


_lut_apply = functools.partial(
    pl.kernel,
    out_type=jax.ShapeDtypeStruct((_B * _C * _N,), jnp.float32),
    mesh=plsc.VectorSubcoreMesh(
        core_axis_name="c", subcore_axis_name="s",
        num_cores=_NUM_CORES, num_subcores=_NUM_SUBCORES),
    compiler_params=pltpu.CompilerParams(needs_layout_passes=False),
    scratch_types=[
        pltpu.VMEM((_LUT_ROW_PAD,), jnp.float32),
        pltpu.VMEM((_C * _CHUNK,), jnp.float32),
        pltpu.VMEM((_C * _CHUNK,), jnp.float32),
    ],
)(_lut_body)


def kernel(image, lut):
    B, C, H, W = image.shape
    img = image.reshape(B * C * H * W)
    lut_flat = lut.reshape(B, _LUT_ROW)
    lut_pad = jnp.pad(lut_flat, ((0, 0), (0, _LUT_ROW_PAD - _LUT_ROW)))
    out = _lut_apply(img, lut_pad.reshape(-1))
    return out.reshape(B, C, H, W)


# speedup vs baseline: 2288.3672x; 1.1465x over previous
import functools

import jax
import jax.numpy as jnp
from jax import lax
from jax.experimental import pallas as pl
from jax.experimental.pallas import tpu as pltpu
from jax.experimental.pallas import tpu_sc as plsc

_NUM_CORES = 2
_NUM_SUBCORES = 16
_LANES = 16
_NW = _NUM_CORES * _NUM_SUBCORES

_B = 8
_C = 3
_N = 512 * 512
_D = 33
_LUT_CH = _D * _D * _D
_LUT_ROW = _C * _LUT_CH
_LUT_BUF = _LUT_ROW + 13
_LUT_TOTAL = _B * _LUT_ROW

_WPB = _NW // _B
_PPW = _N // _WPB
_CHUNK = 1024
_NVEC = _CHUNK // _LANES
_NCHUNK = _PPW // _CHUNK
_NOUTER = _NCHUNK // 2


def _lut_body(img_hbm, lut_hbm, out_hbm, lut_v, in0, in1, out0, out1,
              si0, si1, so):
    cid = lax.axis_index("c")
    sid = lax.axis_index("s")
    wid = sid * _NUM_CORES + cid
    b = wid // _WPB
    w = wid % _WPB

    row_off = b * _LUT_ROW
    start = pl.multiple_of(
        jnp.minimum(row_off - lax.rem(row_off, 8), _LUT_TOTAL - _LUT_BUF), 8)
    d = row_off - start
    pltpu.sync_copy(lut_hbm.at[pl.ds(start, _LUT_BUF)], lut_v)

    base0 = b * _C * _N + w * _PPW

    def issue_in(off, buf, sem):
        for ch in range(_C):
            pltpu.async_copy(img_hbm.at[pl.ds(off + ch * _N, _CHUNK)],
                             buf.at[pl.ds(ch * _CHUNK, _CHUNK)], sem)

    def drain(buf, sem):
        pltpu.make_async_copy(img_hbm.at[pl.ds(0, _C * _CHUNK)], buf,
                              sem).wait()

    def issue_out(off, buf):
        for ch in range(_C):
            pltpu.async_copy(buf.at[pl.ds(ch * _CHUNK, _CHUNK)],
                             out_hbm.at[pl.ds(off + ch * _N, _CHUNK)], so)

    def compute(src, dst):
        @plsc.parallel_loop(0, _NVEC, 1, unroll=2)
        def _vec(j):
            p = j * _LANES
            r = src[pl.ds(p, _LANES)]
            g = src[pl.ds(_CHUNK + p, _LANES)]
            bl = src[pl.ds(2 * _CHUNK + p, _LANES)]
            x = r * 32.0
            y = g * 32.0
            z = bl * 32.0
            x0 = x.astype(jnp.int32)
            y0 = y.astype(jnp.int32)
            z0 = z.astype(jnp.int32)
            wx = x - x0.astype(jnp.float32)
            wy = y - y0.astype(jnp.float32)
            wz = z - z0.astype(jnp.float32)
            idx = (z0 * _D + y0) * _D + (x0 + d)
            i000 = idx
            i001 = idx + 1
            i010 = idx + _D
            i011 = idx + (_D + 1)
            i100 = idx + _D * _D
            i101 = idx + (_D * _D + 1)
            i110 = idx + (_D * _D + _D)
            i111 = idx + (_D * _D + _D + 1)
            ux = 1.0 - wx
            uy = 1.0 - wy
            uz = 1.0 - wz
            a00 = uz * uy
            a01 = uz * wy
            a10 = wz * uy
            a11 = wz * wy
            w000 = a00 * ux
            w001 = a00 * wx
            w010 = a01 * ux
            w011 = a01 * wx
            w100 = a10 * ux
            w101 = a10 * wx
            w110 = a11 * ux
            w111 = a11 * wx
            corners = (
                (i000, w000), (i001, w001), (i010, w010), (i011, w011),
                (i100, w100), (i101, w101), (i110, w110), (i111, w111),
            )
            for ch in range(_C):
                o = ch * _LUT_CH
                t = [plsc.load_gather(lut_v, [iv if o == 0 else iv + o]) * wv
                     for iv, wv in corners]
                acc = (((t[0] + t[1]) + (t[2] + t[3]))
                       + ((t[4] + t[5]) + (t[6] + t[7])))
                dst[pl.ds(ch * _CHUNK + p, _LANES)] = acc

    issue_in(base0, in0, si0)

    @pl.loop(0, _NOUTER)
    def _outer(t):
        off_a = base0 + (2 * t) * _CHUNK
        off_b = off_a + _CHUNK
        issue_in(off_b, in1, si1)
        drain(in0, si0)
        compute(in0, out0)
        issue_out(off_a, out0)

        @pl.when(t < _NOUTER - 1)
        def _():
            issue_in(off_b + _CHUNK, in0, si0)

        drain(in1, si1)
        compute(in1, out1)
        issue_out(off_b, out1)
        drain(out0, so)
        drain(out1, so)


_lut_apply = functools.partial(
    pl.kernel,
    out_type=jax.ShapeDtypeStruct((_B * _C * _N,), jnp.float32),
    mesh=plsc.VectorSubcoreMesh(
        core_axis_name="c", subcore_axis_name="s",
        num_cores=_NUM_CORES, num_subcores=_NUM_SUBCORES),
    compiler_params=pltpu.CompilerParams(needs_layout_passes=False),
    scratch_types=[
        pltpu.VMEM((_LUT_BUF,), jnp.float32),
        pltpu.VMEM((_C * _CHUNK,), jnp.float32),
        pltpu.VMEM((_C * _CHUNK,), jnp.float32),
        pltpu.VMEM((_C * _CHUNK,), jnp.float32),
        pltpu.VMEM((_C * _CHUNK,), jnp.float32),
        pltpu.SemaphoreType.DMA,
        pltpu.SemaphoreType.DMA,
        pltpu.SemaphoreType.DMA,
    ],
)(_lut_body)


def kernel(image, lut):
    B, C, H, W = image.shape
    img = image.reshape(B * C * H * W)
    out = _lut_apply(img, lut.reshape(-1))
    return out.reshape(B, C, H, W)


# speedup vs baseline: 2288.9126x; 1.0002x over previous
import functools

import jax
import jax.numpy as jnp
from jax import lax
from jax.experimental import pallas as pl
from jax.experimental.pallas import tpu as pltpu
from jax.experimental.pallas import tpu_sc as plsc

_NUM_CORES = 2
_NUM_SUBCORES = 16
_LANES = 16
_NW = _NUM_CORES * _NUM_SUBCORES

_B = 8
_C = 3
_N = 512 * 512
_D = 33
_LUT_CH = _D * _D * _D
_LUT_ROW = _C * _LUT_CH
_LUT_BUF = _LUT_ROW + 13
_LUT_TOTAL = _B * _LUT_ROW

_WPB = _NW // _B
_PPW = _N // _WPB
_CHUNK = 1024
_NVEC = _CHUNK // _LANES
_NCHUNK = _PPW // _CHUNK
_NOUTER = _NCHUNK // 2


def _lut_body(img_hbm, lut_hbm, out_hbm, lut_v, in0, in1, out0, out1,
              si0, si1, so):
    cid = lax.axis_index("c")
    sid = lax.axis_index("s")
    wid = sid * _NUM_CORES + cid
    b = wid // _WPB
    w = wid % _WPB

    row_off = b * _LUT_ROW
    start = pl.multiple_of(
        jnp.minimum(row_off - lax.rem(row_off, 8), _LUT_TOTAL - _LUT_BUF), 8)
    d = row_off - start
    pltpu.sync_copy(lut_hbm.at[pl.ds(start, _LUT_BUF)], lut_v)

    base0 = b * _C * _N + w * _PPW

    def issue_in(off, buf, sem):
        for ch in range(_C):
            pltpu.async_copy(img_hbm.at[pl.ds(off + ch * _N, _CHUNK)],
                             buf.at[pl.ds(ch * _CHUNK, _CHUNK)], sem)

    def drain(buf, sem):
        pltpu.make_async_copy(img_hbm.at[pl.ds(0, _C * _CHUNK)], buf,
                              sem).wait()

    def issue_out(off, buf):
        for ch in range(_C):
            pltpu.async_copy(buf.at[pl.ds(ch * _CHUNK, _CHUNK)],
                             out_hbm.at[pl.ds(off + ch * _N, _CHUNK)], so)

    def compute(src, dst):
        @plsc.parallel_loop(0, _NVEC, 1, unroll=2)
        def _vec(j):
            p = j * _LANES
            r = src[pl.ds(p, _LANES)]
            g = src[pl.ds(_CHUNK + p, _LANES)]
            bl = src[pl.ds(2 * _CHUNK + p, _LANES)]
            x = r * 32.0
            y = g * 32.0
            z = bl * 32.0
            x0 = x.astype(jnp.int32)
            y0 = y.astype(jnp.int32)
            z0 = z.astype(jnp.int32)
            wx = x - x0.astype(jnp.float32)
            wy = y - y0.astype(jnp.float32)
            wz = z - z0.astype(jnp.float32)
            idx = (z0 * _D + y0) * _D + (x0 + d)
            i000 = idx
            i001 = idx + 1
            i010 = idx + _D
            i011 = idx + (_D + 1)
            i100 = idx + _D * _D
            i101 = idx + (_D * _D + 1)
            i110 = idx + (_D * _D + _D)
            i111 = idx + (_D * _D + _D + 1)
            ux = 1.0 - wx
            uy = 1.0 - wy
            uz = 1.0 - wz
            a00 = uz * uy
            a01 = uz * wy
            a10 = wz * uy
            a11 = wz * wy
            w000 = a00 * ux
            w001 = a00 * wx
            w010 = a01 * ux
            w011 = a01 * wx
            w100 = a10 * ux
            w101 = a10 * wx
            w110 = a11 * ux
            w111 = a11 * wx
            corners = (
                (i000, w000), (i001, w001), (i010, w010), (i011, w011),
                (i100, w100), (i101, w101), (i110, w110), (i111, w111),
            )
            for ch in range(_C):
                o = ch * _LUT_CH
                t = [plsc.load_gather(lut_v, [iv if o == 0 else iv + o]) * wv
                     for iv, wv in corners]
                acc = (((t[0] + t[1]) + (t[2] + t[3]))
                       + ((t[4] + t[5]) + (t[6] + t[7])))
                dst[pl.ds(ch * _CHUNK + p, _LANES)] = acc

    issue_in(base0, in0, si0)

    @pl.loop(0, _NOUTER)
    def _outer(t):
        off_a = base0 + (2 * t) * _CHUNK
        off_b = off_a + _CHUNK
        issue_in(off_b, in1, si1)

        @pl.when(t > 0)
        def _():
            drain(out0, so)
            drain(out1, so)

        drain(in0, si0)
        compute(in0, out0)
        issue_out(off_a, out0)

        @pl.when(t < _NOUTER - 1)
        def _():
            issue_in(off_b + _CHUNK, in0, si0)

        drain(in1, si1)
        compute(in1, out1)
        issue_out(off_b, out1)

    drain(out0, so)
    drain(out1, so)


_lut_apply = functools.partial(
    pl.kernel,
    out_type=jax.ShapeDtypeStruct((_B * _C * _N,), jnp.float32),
    mesh=plsc.VectorSubcoreMesh(
        core_axis_name="c", subcore_axis_name="s",
        num_cores=_NUM_CORES, num_subcores=_NUM_SUBCORES),
    compiler_params=pltpu.CompilerParams(needs_layout_passes=False),
    scratch_types=[
        pltpu.VMEM((_LUT_BUF,), jnp.float32),
        pltpu.VMEM((_C * _CHUNK,), jnp.float32),
        pltpu.VMEM((_C * _CHUNK,), jnp.float32),
        pltpu.VMEM((_C * _CHUNK,), jnp.float32),
        pltpu.VMEM((_C * _CHUNK,), jnp.float32),
        pltpu.SemaphoreType.DMA,
        pltpu.SemaphoreType.DMA,
        pltpu.SemaphoreType.DMA,
    ],
)(_lut_body)


def kernel(image, lut):
    B, C, H, W = image.shape
    img = image.reshape(B * C * H * W)
    out = _lut_apply(img, lut.reshape(-1))
    return out.reshape(B, C, H, W)


# speedup vs baseline: 2550.1823x; 1.1141x over previous
import functools

import jax
import jax.numpy as jnp
from jax import lax
from jax.experimental import pallas as pl
from jax.experimental.pallas import tpu as pltpu
from jax.experimental.pallas import tpu_sc as plsc

_NUM_CORES = 2
_NUM_SUBCORES = 16
_LANES = 16
_NW = _NUM_CORES * _NUM_SUBCORES

_B = 8
_C = 3
_H = 512
_W = 512
_N = _H * _W
_D = 33
_LUT_CH = _D * _D * _D
_LUT_ROW = _C * _LUT_CH
_LUT_BUF = _LUT_ROW + 13
_LUT_TOTAL = _B * _LUT_ROW

_WPB = _NW // _B
_PPW = _N // _WPB
_RPW = _PPW // _W
_ROWS = 2
_CHUNK = _ROWS * _W
_NVEC = _CHUNK // _LANES
_NCHUNK = _RPW // _ROWS
_NOUTER = _NCHUNK // 2


def _lut_body(img_hbm, lut_hbm, out_hbm,
              lut_v, ina, inb, outa, outb, sia, sib, so):
    cid = lax.axis_index("c")
    sid = lax.axis_index("s")
    wid = sid * _NUM_CORES + cid
    b = wid // _WPB
    w = wid % _WPB

    row_off = b * _LUT_ROW
    start = pl.multiple_of(
        jnp.minimum(row_off - lax.rem(row_off, 8), _LUT_TOTAL - _LUT_BUF), 8)
    d = row_off - start
    pltpu.sync_copy(lut_hbm.at[pl.ds(start, _LUT_BUF)], lut_v)

    row0 = w * _RPW

    def issue_in(row, bufs, sem):
        for ch in range(_C):
            pltpu.async_copy(img_hbm.at[b, ch, pl.ds(row, _ROWS), :],
                             bufs[ch], sem)

    def drain(bufs, sem):
        for ch in range(_C):
            pltpu.make_async_copy(img_hbm.at[0, 0, pl.ds(0, _ROWS), :],
                                  bufs[ch], sem).wait()

    def issue_out(row, bufs):
        for ch in range(_C):
            pltpu.async_copy(bufs[ch],
                             out_hbm.at[b, ch, pl.ds(row, _ROWS), :], so)

    def compute(src, dst):
        @plsc.parallel_loop(0, _NVEC // _ROWS, 1, unroll=2)
        def _vec(j):
            p = j * _LANES
            for rr in range(_ROWS):
                sl = pl.ds(p, _LANES)
                r = src[0][rr, sl]
                g = src[1][rr, sl]
                bl = src[2][rr, sl]
                x = r * 32.0
                y = g * 32.0
                z = bl * 32.0
                x0 = x.astype(jnp.int32)
                y0 = y.astype(jnp.int32)
                z0 = z.astype(jnp.int32)
                wx = x - x0.astype(jnp.float32)
                wy = y - y0.astype(jnp.float32)
                wz = z - z0.astype(jnp.float32)
                idx = (z0 * _D + y0) * _D + (x0 + d)
                i000 = idx
                i001 = idx + 1
                i010 = idx + _D
                i011 = idx + (_D + 1)
                i100 = idx + _D * _D
                i101 = idx + (_D * _D + 1)
                i110 = idx + (_D * _D + _D)
                i111 = idx + (_D * _D + _D + 1)
                ux = 1.0 - wx
                uy = 1.0 - wy
                uz = 1.0 - wz
                a00 = uz * uy
                a01 = uz * wy
                a10 = wz * uy
                a11 = wz * wy
                w000 = a00 * ux
                w001 = a00 * wx
                w010 = a01 * ux
                w011 = a01 * wx
                w100 = a10 * ux
                w101 = a10 * wx
                w110 = a11 * ux
                w111 = a11 * wx
                corners = (
                    (i000, w000), (i001, w001), (i010, w010), (i011, w011),
                    (i100, w100), (i101, w101), (i110, w110), (i111, w111),
                )
                for ch in range(_C):
                    o = ch * _LUT_CH
                    t = [plsc.load_gather(lut_v,
                                          [iv if o == 0 else iv + o]) * wv
                         for iv, wv in corners]
                    acc = (((t[0] + t[1]) + (t[2] + t[3]))
                           + ((t[4] + t[5]) + (t[6] + t[7])))
                    dst[ch][rr, sl] = acc

    issue_in(row0, ina, sia)

    @pl.loop(0, _NOUTER)
    def _outer(t):
        row_a = row0 + (2 * t) * _ROWS
        row_b = row_a + _ROWS
        issue_in(row_b, inb, sib)

        @pl.when(t > 0)
        def _():
            drain(outa, so)
            drain(outb, so)

        drain(ina, sia)
        compute(ina, outa)
        issue_out(row_a, outa)

        @pl.when(t < _NOUTER - 1)
        def _():
            issue_in(row_b + _ROWS, ina, sia)

        drain(inb, sib)
        compute(inb, outb)
        issue_out(row_b, outb)

    drain(outa, so)
    drain(outb, so)


def _vmem_chunk():
    return pltpu.VMEM((_ROWS, _W), jnp.float32)


_lut_apply = functools.partial(
    pl.kernel,
    out_type=jax.ShapeDtypeStruct((_B, _C, _H, _W), jnp.float32),
    mesh=plsc.VectorSubcoreMesh(
        core_axis_name="c", subcore_axis_name="s",
        num_cores=_NUM_CORES, num_subcores=_NUM_SUBCORES),
    compiler_params=pltpu.CompilerParams(needs_layout_passes=False),
    scratch_types=[
        pltpu.VMEM((_LUT_BUF,), jnp.float32),
        [_vmem_chunk() for _ in range(_C)],
        [_vmem_chunk() for _ in range(_C)],
        [_vmem_chunk() for _ in range(_C)],
        [_vmem_chunk() for _ in range(_C)],
        pltpu.SemaphoreType.DMA,
        pltpu.SemaphoreType.DMA,
        pltpu.SemaphoreType.DMA,
    ],
)(_lut_body)


def kernel(image, lut):
    return _lut_apply(image, lut.reshape(-1))


# speedup vs baseline: 3457.8408x; 1.3559x over previous
import functools

import jax
import jax.numpy as jnp
from jax import lax
from jax.experimental import pallas as pl
from jax.experimental.pallas import tpu as pltpu
from jax.experimental.pallas import tpu_sc as plsc

_NUM_CORES = 2
_NUM_SUBCORES = 16
_LANES = 16
_NW = _NUM_CORES * _NUM_SUBCORES

_B = 8
_C = 3
_H = 512
_W = 512
_N = _H * _W
_D = 33
_LUT_CH = _D * _D * _D
_LUT_ROW = _C * _LUT_CH
_LUT_BUF = _LUT_ROW + 13
_LUT_TOTAL = _B * _LUT_ROW

_WPB = _NW // _B
_PPW = _N // _WPB
_RPW = _PPW // _W
_ROWS = 2
_CHUNK = _ROWS * _W
_NVEC = _CHUNK // _LANES
_NCHUNK = _RPW // _ROWS
_NOUTER = _NCHUNK // 2


def _lut_body(img_hbm, lut_hbm, out_hbm,
              lut_v, ina, inb, outa, outb, sia, sib, so):
    cid = lax.axis_index("c")
    sid = lax.axis_index("s")
    wid = sid * _NUM_CORES + cid
    b = wid // _WPB
    w = wid % _WPB

    row_off = b * _LUT_ROW
    start = pl.multiple_of(
        jnp.minimum(row_off - lax.rem(row_off, 8), _LUT_TOTAL - _LUT_BUF), 8)
    d = row_off - start
    pltpu.sync_copy(lut_hbm.at[pl.ds(start, _LUT_BUF)], lut_v)

    row0 = w * _RPW

    def issue_in(row, bufs, sem):
        for ch in range(_C):
            pltpu.async_copy(img_hbm.at[b, ch, pl.ds(row, _ROWS), :],
                             bufs[ch], sem)

    def drain(bufs, sem):
        for ch in range(_C):
            pltpu.make_async_copy(img_hbm.at[0, 0, pl.ds(0, _ROWS), :],
                                  bufs[ch], sem).wait()

    def issue_out(row, bufs):
        for ch in range(_C):
            pltpu.async_copy(bufs[ch],
                             out_hbm.at[b, ch, pl.ds(row, _ROWS), :], so)

    def compute(src, dst):
        @plsc.parallel_loop(0, _NVEC // _ROWS, 1, unroll=1)
        def _vec(j):
            p = j * _LANES
            for rr in range(_ROWS):
                sl = pl.ds(p, _LANES)
                r = src[0][rr, sl]
                g = src[1][rr, sl]
                bl = src[2][rr, sl]
                x = r * 32.0
                y = g * 32.0
                z = bl * 32.0
                x0 = x.astype(jnp.int32)
                y0 = y.astype(jnp.int32)
                z0 = z.astype(jnp.int32)
                wx = x - x0.astype(jnp.float32)
                wy = y - y0.astype(jnp.float32)
                wz = z - z0.astype(jnp.float32)
                idx = (z0 * _D + y0) * _D + (x0 + d)
                i000 = idx
                i001 = idx + 1
                i010 = idx + _D
                i011 = idx + (_D + 1)
                i100 = idx + _D * _D
                i101 = idx + (_D * _D + 1)
                i110 = idx + (_D * _D + _D)
                i111 = idx + (_D * _D + _D + 1)
                ux = 1.0 - wx
                uy = 1.0 - wy
                uz = 1.0 - wz
                a00 = uz * uy
                a01 = uz * wy
                a10 = wz * uy
                a11 = wz * wy
                w000 = a00 * ux
                w001 = a00 * wx
                w010 = a01 * ux
                w011 = a01 * wx
                w100 = a10 * ux
                w101 = a10 * wx
                w110 = a11 * ux
                w111 = a11 * wx
                corners = (
                    (i000, w000), (i001, w001), (i010, w010), (i011, w011),
                    (i100, w100), (i101, w101), (i110, w110), (i111, w111),
                )
                for ch in range(_C):
                    o = ch * _LUT_CH
                    t = [plsc.load_gather(lut_v,
                                          [iv if o == 0 else iv + o]) * wv
                         for iv, wv in corners]
                    acc = (((t[0] + t[1]) + (t[2] + t[3]))
                           + ((t[4] + t[5]) + (t[6] + t[7])))
                    dst[ch][rr, sl] = acc

    issue_in(row0, ina, sia)

    @pl.loop(0, _NOUTER)
    def _outer(t):
        row_a = row0 + (2 * t) * _ROWS
        row_b = row_a + _ROWS
        issue_in(row_b, inb, sib)

        @pl.when(t > 0)
        def _():
            drain(outa, so)
            drain(outb, so)

        drain(ina, sia)
        compute(ina, outa)
        issue_out(row_a, outa)

        @pl.when(t < _NOUTER - 1)
        def _():
            issue_in(row_b + _ROWS, ina, sia)

        drain(inb, sib)
        compute(inb, outb)
        issue_out(row_b, outb)

    drain(outa, so)
    drain(outb, so)


def _vmem_chunk():
    return pltpu.VMEM((_ROWS, _W), jnp.float32)


_lut_apply = functools.partial(
    pl.kernel,
    out_type=jax.ShapeDtypeStruct((_B, _C, _H, _W), jnp.float32),
    mesh=plsc.VectorSubcoreMesh(
        core_axis_name="c", subcore_axis_name="s",
        num_cores=_NUM_CORES, num_subcores=_NUM_SUBCORES),
    compiler_params=pltpu.CompilerParams(needs_layout_passes=False),
    scratch_types=[
        pltpu.VMEM((_LUT_BUF,), jnp.float32),
        [_vmem_chunk() for _ in range(_C)],
        [_vmem_chunk() for _ in range(_C)],
        [_vmem_chunk() for _ in range(_C)],
        [_vmem_chunk() for _ in range(_C)],
        pltpu.SemaphoreType.DMA,
        pltpu.SemaphoreType.DMA,
        pltpu.SemaphoreType.DMA,
    ],
)(_lut_body)


def kernel(image, lut):
    return _lut_apply(image, lut.reshape(-1))


# speedup vs baseline: 3469.9164x; 1.0035x over previous
import functools

import jax
import jax.numpy as jnp
from jax import lax
from jax.experimental import pallas as pl
from jax.experimental.pallas import tpu as pltpu
from jax.experimental.pallas import tpu_sc as plsc

_NUM_CORES = 2
_NUM_SUBCORES = 16
_LANES = 16
_NW = _NUM_CORES * _NUM_SUBCORES

_B = 8
_C = 3
_H = 512
_W = 512
_N = _H * _W
_D = 33
_LUT_CH = _D * _D * _D
_LUT_ROW = _C * _LUT_CH
_LUT_BUF = _LUT_ROW + 13
_LUT_TOTAL = _B * _LUT_ROW

_WPB = _NW // _B
_PPW = _N // _WPB
_RPW = _PPW // _W
_ROWS = 2
_CHUNK = _ROWS * _W
_NVEC = _CHUNK // _LANES
_NCHUNK = _RPW // _ROWS
_NOUTER = _NCHUNK // 2


def _lut_body(img_hbm, lut_hbm, out_hbm,
              lut_v, ina, inb, outa, outb, sia, sib, so):
    cid = lax.axis_index("c")
    sid = lax.axis_index("s")
    wid = sid * _NUM_CORES + cid
    b = wid // _WPB
    w = wid % _WPB

    row_off = b * _LUT_ROW
    start = pl.multiple_of(
        jnp.minimum(row_off - lax.rem(row_off, 8), _LUT_TOTAL - _LUT_BUF), 8)
    d = row_off - start

    row0 = w * _RPW

    def issue_in(row, bufs, sem):
        for ch in range(_C):
            pltpu.async_copy(img_hbm.at[b, ch, pl.ds(row, _ROWS), :],
                             bufs[ch], sem)

    def drain(bufs, sem):
        for ch in range(_C):
            pltpu.make_async_copy(img_hbm.at[0, 0, pl.ds(0, _ROWS), :],
                                  bufs[ch], sem).wait()

    def issue_out(row, bufs):
        for ch in range(_C):
            pltpu.async_copy(bufs[ch],
                             out_hbm.at[b, ch, pl.ds(row, _ROWS), :], so)

    def compute(src, dst):
        @plsc.parallel_loop(0, _NVEC // _ROWS, 1, unroll=1)
        def _vec(j):
            p = j * _LANES
            for rr in range(_ROWS):
                sl = pl.ds(p, _LANES)
                r = src[0][rr, sl]
                g = src[1][rr, sl]
                bl = src[2][rr, sl]
                x = r * 32.0
                y = g * 32.0
                z = bl * 32.0
                x0 = x.astype(jnp.int32)
                y0 = y.astype(jnp.int32)
                z0 = z.astype(jnp.int32)
                wx = x - x0.astype(jnp.float32)
                wy = y - y0.astype(jnp.float32)
                wz = z - z0.astype(jnp.float32)
                idx = (z0 * _D + y0) * _D + (x0 + d)
                i000 = idx
                i001 = idx + 1
                i010 = idx + _D
                i011 = idx + (_D + 1)
                i100 = idx + _D * _D
                i101 = idx + (_D * _D + 1)
                i110 = idx + (_D * _D + _D)
                i111 = idx + (_D * _D + _D + 1)
                ux = 1.0 - wx
                uy = 1.0 - wy
                uz = 1.0 - wz
                a00 = uz * uy
                a01 = uz * wy
                a10 = wz * uy
                a11 = wz * wy
                w000 = a00 * ux
                w001 = a00 * wx
                w010 = a01 * ux
                w011 = a01 * wx
                w100 = a10 * ux
                w101 = a10 * wx
                w110 = a11 * ux
                w111 = a11 * wx
                corners = (
                    (i000, w000), (i001, w001), (i010, w010), (i011, w011),
                    (i100, w100), (i101, w101), (i110, w110), (i111, w111),
                )
                for ch in range(_C):
                    o = ch * _LUT_CH
                    t = [plsc.load_gather(lut_v,
                                          [iv if o == 0 else iv + o]) * wv
                         for iv, wv in corners]
                    acc = (((t[0] + t[1]) + (t[2] + t[3]))
                           + ((t[4] + t[5]) + (t[6] + t[7])))
                    dst[ch][rr, sl] = acc

    issue_in(row0, ina, sia)
    pltpu.sync_copy(lut_hbm.at[pl.ds(start, _LUT_BUF)], lut_v)

    @pl.loop(0, _NOUTER)
    def _outer(t):
        row_a = row0 + (2 * t) * _ROWS
        row_b = row_a + _ROWS
        issue_in(row_b, inb, sib)

        @pl.when(t > 0)
        def _():
            drain(outa, so)
            drain(outb, so)

        drain(ina, sia)
        compute(ina, outa)
        issue_out(row_a, outa)

        @pl.when(t < _NOUTER - 1)
        def _():
            issue_in(row_b + _ROWS, ina, sia)

        drain(inb, sib)
        compute(inb, outb)
        issue_out(row_b, outb)

    drain(outa, so)
    drain(outb, so)


def _vmem_chunk():
    return pltpu.VMEM((_ROWS, _W), jnp.float32)


_lut_apply = functools.partial(
    pl.kernel,
    out_type=jax.ShapeDtypeStruct((_B, _C, _H, _W), jnp.float32),
    mesh=plsc.VectorSubcoreMesh(
        core_axis_name="c", subcore_axis_name="s",
        num_cores=_NUM_CORES, num_subcores=_NUM_SUBCORES),
    compiler_params=pltpu.CompilerParams(needs_layout_passes=False),
    scratch_types=[
        pltpu.VMEM((_LUT_BUF,), jnp.float32),
        [_vmem_chunk() for _ in range(_C)],
        [_vmem_chunk() for _ in range(_C)],
        [_vmem_chunk() for _ in range(_C)],
        [_vmem_chunk() for _ in range(_C)],
        pltpu.SemaphoreType.DMA,
        pltpu.SemaphoreType.DMA,
        pltpu.SemaphoreType.DMA,
    ],
)(_lut_body)


def kernel(image, lut):
    return _lut_apply(image, lut.reshape(-1))


# speedup vs baseline: 3574.3824x; 1.0301x over previous
import functools

import jax
import jax.numpy as jnp
from jax import lax
from jax.experimental import pallas as pl
from jax.experimental.pallas import tpu as pltpu
from jax.experimental.pallas import tpu_sc as plsc

_NUM_CORES = 2
_NUM_SUBCORES = 16
_LANES = 16
_NW = _NUM_CORES * _NUM_SUBCORES

_B = 8
_C = 3
_H = 512
_W = 512
_N = _H * _W
_D = 33
_LUT_CH = _D * _D * _D
_LUT_ROW = _C * _LUT_CH
_LUT_WIN = _LUT_ROW + 13
_LUT_TOTAL = _B * _LUT_ROW
_LUT_SLICE = _LUT_CH + 8
_MAX_OFF = 2 * _LUT_CH + (_D * _D + _D + 1)
_LUT_BUF = (_MAX_OFF - _MAX_OFF % 8) + _LUT_SLICE + 7

_WPB = _NW // _B
_PPW = _N // _WPB
_RPW = _PPW // _W
_ROWS = 2
_CHUNK = _ROWS * _W
_NVEC = _CHUNK // _LANES
_NCHUNK = _RPW // _ROWS
_NOUTER = _NCHUNK // 2


def _lut_body(img_hbm, lut_hbm, out_hbm,
              lut_v, ina, inb, outa, outb, sia, sib, so):
    cid = lax.axis_index("c")
    sid = lax.axis_index("s")
    wid = sid * _NUM_CORES + cid
    b = wid // _WPB
    w = wid % _WPB

    row_off = b * _LUT_ROW
    start = pl.multiple_of(
        jnp.minimum(row_off - lax.rem(row_off, 8), _LUT_TOTAL - _LUT_WIN), 8)
    d = row_off - start

    row0 = w * _RPW

    def issue_in(row, bufs, sem):
        for ch in range(_C):
            pltpu.async_copy(img_hbm.at[b, ch, pl.ds(row, _ROWS), :],
                             bufs[ch], sem)

    def drain(bufs, sem):
        for ch in range(_C):
            pltpu.make_async_copy(img_hbm.at[0, 0, pl.ds(0, _ROWS), :],
                                  bufs[ch], sem).wait()

    def issue_out(row, bufs):
        for ch in range(_C):
            pltpu.async_copy(bufs[ch],
                             out_hbm.at[b, ch, pl.ds(row, _ROWS), :], so)

    def compute(src, dst):
        @plsc.parallel_loop(0, _NVEC // _ROWS, 1, unroll=1)
        def _vec(j):
            p = j * _LANES
            for rr in range(_ROWS):
                sl = pl.ds(p, _LANES)
                r = src[0][rr, sl]
                g = src[1][rr, sl]
                bl = src[2][rr, sl]
                x = r * 32.0
                y = g * 32.0
                z = bl * 32.0
                x0 = x.astype(jnp.int32)
                y0 = y.astype(jnp.int32)
                z0 = z.astype(jnp.int32)
                wx = x - x0.astype(jnp.float32)
                wy = y - y0.astype(jnp.float32)
                wz = z - z0.astype(jnp.float32)
                idx = (z0 * _D + y0) * _D + (x0 + d)
                ux = 1.0 - wx
                uy = 1.0 - wy
                uz = 1.0 - wz
                a00 = uz * uy
                a01 = uz * wy
                a10 = wz * uy
                a11 = wz * wy
                w000 = a00 * ux
                w001 = a00 * wx
                w010 = a01 * ux
                w011 = a01 * wx
                w100 = a10 * ux
                w101 = a10 * wx
                w110 = a11 * ux
                w111 = a11 * wx
                corners = (
                    (0, w000), (1, w001), (_D, w010), (_D + 1, w011),
                    (_D * _D, w100), (_D * _D + 1, w101),
                    (_D * _D + _D, w110), (_D * _D + _D + 1, w111),
                )
                idx_r = {0: idx}
                for ch in range(_C):
                    o = ch * _LUT_CH
                    t = []
                    for co, wv in corners:
                        full = o + co
                        res = full % 8
                        if res not in idx_r:
                            idx_r[res] = idx + res
                        g = plsc.load_gather(
                            lut_v.at[pl.ds(full - res, _LUT_SLICE)],
                            [idx_r[res]])
                        t.append(g * wv)
                    acc = (((t[0] + t[1]) + (t[2] + t[3]))
                           + ((t[4] + t[5]) + (t[6] + t[7])))
                    dst[ch][rr, sl] = acc

    issue_in(row0, ina, sia)
    pltpu.sync_copy(lut_hbm.at[pl.ds(start, _LUT_WIN)],
                    lut_v.at[pl.ds(0, _LUT_WIN)])

    @pl.loop(0, _NOUTER)
    def _outer(t):
        row_a = row0 + (2 * t) * _ROWS
        row_b = row_a + _ROWS
        issue_in(row_b, inb, sib)

        @pl.when(t > 0)
        def _():
            drain(outa, so)
            drain(outb, so)

        drain(ina, sia)
        compute(ina, outa)
        issue_out(row_a, outa)

        @pl.when(t < _NOUTER - 1)
        def _():
            issue_in(row_b + _ROWS, ina, sia)

        drain(inb, sib)
        compute(inb, outb)
        issue_out(row_b, outb)

    drain(outa, so)
    drain(outb, so)


def _vmem_chunk():
    return pltpu.VMEM((_ROWS, _W), jnp.float32)


_lut_apply = functools.partial(
    pl.kernel,
    out_type=jax.ShapeDtypeStruct((_B, _C, _H, _W), jnp.float32),
    mesh=plsc.VectorSubcoreMesh(
        core_axis_name="c", subcore_axis_name="s",
        num_cores=_NUM_CORES, num_subcores=_NUM_SUBCORES),
    compiler_params=pltpu.CompilerParams(needs_layout_passes=False),
    scratch_types=[
        pltpu.VMEM((_LUT_BUF,), jnp.float32),
        [_vmem_chunk() for _ in range(_C)],
        [_vmem_chunk() for _ in range(_C)],
        [_vmem_chunk() for _ in range(_C)],
        [_vmem_chunk() for _ in range(_C)],
        pltpu.SemaphoreType.DMA,
        pltpu.SemaphoreType.DMA,
        pltpu.SemaphoreType.DMA,
    ],
)(_lut_body)


def kernel(image, lut):
    return _lut_apply(image, lut.reshape(-1))
